# speedup vs baseline: 14.7986x; 14.7986x over previous
import functools

import jax
import jax.numpy as jnp
from jax import lax
from jax.experimental import pallas as pl
from jax.experimental.pallas import tpu as pltpu
from jax.experimental.pallas import tpu_sc as plsc

N = 10000
D = 128
N_PAD = 10240
DUMMY = N
NC = 2
NS = 16
NW = NC * NS
CHUNK = 128
ROWS_PER_SUB = N_PAD // NS
C_U = 1.0
C_SIGMA = 2.0
COEF = (C_SIGMA / D) ** 0.5

_MESH = plsc.VectorSubcoreMesh(core_axis_name="c", subcore_axis_name="s")


@functools.partial(
    pl.kernel,
    out_type=jax.ShapeDtypeStruct((NC * N_PAD, 16), jnp.float32),
    mesh=_MESH,
    scratch_types=[
        pltpu.VMEM((CHUNK,), jnp.int32),
        pltpu.VMEM((CHUNK, 16), jnp.float32),
        pltpu.VMEM((CHUNK, 16), jnp.float32),
        pltpu.VMEM_SHARED((N_PAD, 16), jnp.float32),
    ],
)
def _sc_deg(row_hbm, deg_hbm, idx_v, ones_v, zero_v, deg_sh):
    c = lax.axis_index("c")
    s = lax.axis_index("s")
    w = c * NS + s
    e_pad = row_hbm.shape[0]
    edges_per_tile = e_pad // NW
    chunks_per_tile = edges_per_tile // CHUNK

    @pl.loop(0, CHUNK)
    def _fill(i):
        ones_v[i, :] = jnp.full((16,), 1.0, jnp.float32)
        zero_v[i, :] = jnp.zeros((16,), jnp.float32)

    @pl.loop(0, ROWS_PER_SUB // CHUNK)
    def _zero(j):
        pltpu.sync_copy(zero_v, deg_sh.at[pl.ds(s * ROWS_PER_SUB + j * CHUNK, CHUNK)])

    plsc.subcore_barrier()

    base = w * edges_per_tile

    @pl.loop(0, chunks_per_tile)
    def _hist(ch):
        pltpu.sync_copy(row_hbm.at[pl.ds(base + ch * CHUNK, CHUNK)], idx_v)
        pltpu.sync_copy(ones_v, deg_sh.at[idx_v], add=True)

    plsc.subcore_barrier()
    pltpu.sync_copy(
        deg_sh.at[pl.ds(s * ROWS_PER_SUB, ROWS_PER_SUB)],
        deg_hbm.at[pl.ds(c * N_PAD + s * ROWS_PER_SUB, ROWS_PER_SUB)],
    )


@functools.partial(
    pl.kernel,
    out_type=jax.ShapeDtypeStruct((NC * N_PAD, D), jnp.float32),
    mesh=_MESH,
    scratch_types=[
        pltpu.VMEM((CHUNK,), jnp.int32),
        pltpu.VMEM((CHUNK,), jnp.int32),
        pltpu.VMEM((CHUNK, D), jnp.float32),
        pltpu.VMEM_SHARED((N_PAD, D), jnp.float32),
    ],
)
def _sc_aggr(h2_hbm, row_hbm, col_hbm, acc_hbm, ridx_v, cidx_v, msg_v, acc_sh):
    c = lax.axis_index("c")
    s = lax.axis_index("s")
    w = c * NS + s
    e_pad = row_hbm.shape[0]
    edges_per_tile = e_pad // NW
    chunks_per_tile = edges_per_tile // CHUNK

    @pl.loop(0, CHUNK)
    def _zero_buf(i):
        for j in range(D // 16):
            msg_v[i, pl.ds(j * 16, 16)] = jnp.zeros((16,), jnp.float32)

    @pl.loop(0, ROWS_PER_SUB // CHUNK)
    def _zero_acc(j):
        pltpu.sync_copy(msg_v, acc_sh.at[pl.ds(s * ROWS_PER_SUB + j * CHUNK, CHUNK)])

    plsc.subcore_barrier()

    base = w * edges_per_tile

    @pl.loop(0, chunks_per_tile)
    def _edges(ch):
        off = base + ch * CHUNK
        pltpu.sync_copy(row_hbm.at[pl.ds(off, CHUNK)], ridx_v)
        pltpu.sync_copy(col_hbm.at[pl.ds(off, CHUNK)], cidx_v)
        pltpu.sync_copy(h2_hbm.at[ridx_v], msg_v)
        pltpu.sync_copy(msg_v, acc_sh.at[cidx_v], add=True)

    plsc.subcore_barrier()
    pltpu.sync_copy(
        acc_sh.at[pl.ds(s * ROWS_PER_SUB, ROWS_PER_SUB)],
        acc_hbm.at[pl.ds(c * N_PAD + s * ROWS_PER_SUB, ROWS_PER_SUB)],
    )


_BLK = 1024
_NBLK = N_PAD // _BLK


def _h2_body(x_ref, w_ref, b_ref, d0_ref, d1_ref, o_ref):
    deg = d0_ref[:, 0:1] + d1_ref[:, 0:1] + 1.0
    dis = lax.rsqrt(deg)
    h = lax.dot_general(
        x_ref[...], w_ref[...], (((1,), (1,)), ((), ())),
        preferred_element_type=jnp.float32,
    )
    o_ref[...] = (h + b_ref[...]) * dis


def _tc_h2(x_pad, w, b2, degp):
    return pl.pallas_call(
        _h2_body,
        grid=(_NBLK,),
        in_specs=[
            pl.BlockSpec((_BLK, D), lambda i: (i, 0)),
            pl.BlockSpec((D, D), lambda i: (0, 0)),
            pl.BlockSpec((1, D), lambda i: (0, 0)),
            pl.BlockSpec((_BLK, 16), lambda i: (i, 0)),
            pl.BlockSpec((_BLK, 16), lambda i: (_NBLK + i, 0)),
        ],
        out_specs=pl.BlockSpec((_BLK, D), lambda i: (i, 0)),
        out_shape=jax.ShapeDtypeStruct((N_PAD, D), jnp.float32),
    )(x_pad, w, b2, degp, degp)


def _out_body(a0_ref, a1_ref, h2_ref, d0_ref, d1_ref, o_ref):
    deg = d0_ref[:, 0:1] + d1_ref[:, 0:1] + 1.0
    dis = lax.rsqrt(deg)
    ssum = a0_ref[...] + a1_ref[...] + h2_ref[...]
    o_ref[...] = COEF * jnp.maximum(C_U * ssum * dis, 0.0)


def _tc_out(acc, h2, degp):
    return pl.pallas_call(
        _out_body,
        grid=(_NBLK,),
        in_specs=[
            pl.BlockSpec((_BLK, D), lambda i: (i, 0)),
            pl.BlockSpec((_BLK, D), lambda i: (_NBLK + i, 0)),
            pl.BlockSpec((_BLK, D), lambda i: (i, 0)),
            pl.BlockSpec((_BLK, 16), lambda i: (i, 0)),
            pl.BlockSpec((_BLK, 16), lambda i: (_NBLK + i, 0)),
        ],
        out_specs=pl.BlockSpec((_BLK, D), lambda i: (i, 0)),
        out_shape=jax.ShapeDtypeStruct((N_PAD, D), jnp.float32),
    )(acc, acc, h2, degp, degp)


def kernel(x, edge_index, W, b):
    e = edge_index.shape[1]
    e_pad = ((e + NW * CHUNK - 1) // (NW * CHUNK)) * (NW * CHUNK)
    pad = e_pad - e
    row = jnp.concatenate([edge_index[0], jnp.full((pad,), DUMMY, jnp.int32)])
    col = jnp.concatenate([edge_index[1], jnp.full((pad,), DUMMY, jnp.int32)])
    x_pad = jnp.concatenate([x, jnp.zeros((N_PAD - N, D), jnp.float32)])

    degp = _sc_deg(row)
    h2 = _tc_h2(x_pad, W, b.reshape(1, D), degp)
    acc = _sc_aggr(h2, row, col)
    out = _tc_out(acc, h2, degp)
    return out[:N]


# speedup vs baseline: 20.8142x; 1.4065x over previous
import functools

import jax
import jax.numpy as jnp
from jax import lax
from jax.experimental import pallas as pl
from jax.experimental.pallas import tpu as pltpu
from jax.experimental.pallas import tpu_sc as plsc

N = 10000
D = 128
N_PAD = 10240
DUMMY = N
NC = 2
NS = 16
NW = NC * NS
CHUNK = 128
ROWS_PER_SUB = N_PAD // NS
C_U = 1.0
C_SIGMA = 2.0
COEF = (C_SIGMA / D) ** 0.5

_MESH = plsc.VectorSubcoreMesh(core_axis_name="c", subcore_axis_name="s")


@functools.partial(
    pl.kernel,
    out_type=jax.ShapeDtypeStruct((NC * N_PAD, 16), jnp.float32),
    mesh=_MESH,
    scratch_types=[
        pltpu.VMEM((CHUNK,), jnp.int32),
        pltpu.VMEM((CHUNK, 16), jnp.float32),
        pltpu.VMEM((CHUNK, 16), jnp.float32),
        pltpu.VMEM_SHARED((N_PAD, 16), jnp.float32),
    ],
)
def _sc_deg(row_hbm, deg_hbm, idx_v, ones_v, zero_v, deg_sh):
    c = lax.axis_index("c")
    s = lax.axis_index("s")
    w = c * NS + s
    e_pad = row_hbm.shape[0]
    edges_per_tile = e_pad // NW
    chunks_per_tile = edges_per_tile // CHUNK

    @pl.loop(0, CHUNK)
    def _fill(i):
        ones_v[i, :] = jnp.full((16,), 1.0, jnp.float32)
        zero_v[i, :] = jnp.zeros((16,), jnp.float32)

    @pl.loop(0, ROWS_PER_SUB // CHUNK)
    def _zero(j):
        pltpu.sync_copy(zero_v, deg_sh.at[pl.ds(s * ROWS_PER_SUB + j * CHUNK, CHUNK)])

    plsc.subcore_barrier()

    base = w * edges_per_tile

    @pl.loop(0, chunks_per_tile)
    def _hist(ch):
        pltpu.sync_copy(row_hbm.at[pl.ds(base + ch * CHUNK, CHUNK)], idx_v)
        pltpu.sync_copy(ones_v, deg_sh.at[idx_v], add=True)

    plsc.subcore_barrier()
    pltpu.sync_copy(
        deg_sh.at[pl.ds(s * ROWS_PER_SUB, ROWS_PER_SUB)],
        deg_hbm.at[pl.ds(c * N_PAD + s * ROWS_PER_SUB, ROWS_PER_SUB)],
    )


@functools.partial(
    pl.kernel,
    out_type=jax.ShapeDtypeStruct((NC * N_PAD, D), jnp.float32),
    mesh=_MESH,
    scratch_types=[
        pltpu.VMEM((CHUNK,), jnp.int32),
        pltpu.VMEM((CHUNK,), jnp.int32),
        pltpu.VMEM((CHUNK, D), jnp.float32),
        pltpu.VMEM_SHARED((N_PAD, D), jnp.float32),
    ],
)
def _sc_aggr(h2_hbm, row_hbm, col_hbm, acc_hbm, ridx_v, cidx_v, msg_v, acc_sh):
    c = lax.axis_index("c")
    s = lax.axis_index("s")
    w = c * NS + s
    e_pad = row_hbm.shape[0]
    edges_per_tile = e_pad // NW
    chunks_per_tile = edges_per_tile // CHUNK

    @pl.loop(0, CHUNK)
    def _zero_buf(i):
        for j in range(D // 16):
            msg_v[i, pl.ds(j * 16, 16)] = jnp.zeros((16,), jnp.float32)

    @pl.loop(0, ROWS_PER_SUB // CHUNK)
    def _zero_acc(j):
        pltpu.sync_copy(msg_v, acc_sh.at[pl.ds(s * ROWS_PER_SUB + j * CHUNK, CHUNK)])

    plsc.subcore_barrier()

    base = w * edges_per_tile

    @pl.loop(0, chunks_per_tile)
    def _edges(ch):
        off = base + ch * CHUNK
        pltpu.sync_copy(row_hbm.at[pl.ds(off, CHUNK)], ridx_v)
        pltpu.sync_copy(col_hbm.at[pl.ds(off, CHUNK)], cidx_v)
        pltpu.sync_copy(h2_hbm.at[ridx_v], msg_v)
        pltpu.sync_copy(msg_v, acc_sh.at[cidx_v], add=True)

    plsc.subcore_barrier()
    pltpu.sync_copy(
        acc_sh.at[pl.ds(s * ROWS_PER_SUB, ROWS_PER_SUB)],
        acc_hbm.at[pl.ds(c * N_PAD + s * ROWS_PER_SUB, ROWS_PER_SUB)],
    )


_BLK = 1024
_NBLK = N_PAD // _BLK


def _h2_body(x_ref, w_ref, b_ref, d0_ref, d1_ref, o_ref):
    deg = d0_ref[:, 0:1] + d1_ref[:, 0:1] + 1.0
    dis = lax.rsqrt(deg)
    h = lax.dot_general(
        x_ref[...], w_ref[...], (((1,), (1,)), ((), ())),
        preferred_element_type=jnp.float32,
    )
    o_ref[...] = (h + b_ref[...]) * dis


def _tc_h2(x_pad, w, b2, degp):
    return pl.pallas_call(
        _h2_body,
        grid=(_NBLK,),
        in_specs=[
            pl.BlockSpec((_BLK, D), lambda i: (i, 0)),
            pl.BlockSpec((D, D), lambda i: (0, 0)),
            pl.BlockSpec((1, D), lambda i: (0, 0)),
            pl.BlockSpec((_BLK, 16), lambda i: (i, 0)),
            pl.BlockSpec((_BLK, 16), lambda i: (_NBLK + i, 0)),
        ],
        out_specs=pl.BlockSpec((_BLK, D), lambda i: (i, 0)),
        out_shape=jax.ShapeDtypeStruct((N_PAD, D), jnp.float32),
    )(x_pad, w, b2, degp, degp)


def _out_body(a0_ref, a1_ref, h2_ref, d0_ref, d1_ref, o_ref):
    deg = d0_ref[:, 0:1] + d1_ref[:, 0:1] + 1.0
    dis = lax.rsqrt(deg)
    ssum = a0_ref[...] + a1_ref[...] + h2_ref[...]
    o_ref[...] = COEF * jnp.maximum(C_U * ssum * dis, 0.0)


def _tc_out(acc, h2, degp):
    return pl.pallas_call(
        _out_body,
        grid=(_NBLK,),
        in_specs=[
            pl.BlockSpec((_BLK, D), lambda i: (i, 0)),
            pl.BlockSpec((_BLK, D), lambda i: (_NBLK + i, 0)),
            pl.BlockSpec((_BLK, D), lambda i: (i, 0)),
            pl.BlockSpec((_BLK, 16), lambda i: (i, 0)),
            pl.BlockSpec((_BLK, 16), lambda i: (_NBLK + i, 0)),
        ],
        out_specs=pl.BlockSpec((_BLK, D), lambda i: (i, 0)),
        out_shape=jax.ShapeDtypeStruct((N_PAD, D), jnp.float32),
    )(acc, acc, h2, degp, degp)


def kernel(x, edge_index, W, b):
    e = edge_index.shape[1]
    e_pad = ((e + NW * CHUNK - 1) // (NW * CHUNK)) * (NW * CHUNK)
    pad = e_pad - e
    pad_idx = DUMMY + (jnp.arange(pad, dtype=jnp.int32) % (N_PAD - N))
    row = jnp.concatenate([edge_index[0], pad_idx])
    col = jnp.concatenate([edge_index[1], pad_idx])
    x_pad = jnp.concatenate([x, jnp.zeros((N_PAD - N, D), jnp.float32)])

    degp = _sc_deg(row)
    h2 = _tc_h2(x_pad, W, b.reshape(1, D), degp)
    acc = _sc_aggr(h2, row, col)
    out = _tc_out(acc, h2, degp)
    return out[:N]
